# baseline (speedup 1.0000x reference)
import functools

import jax
import jax.numpy as jnp
from jax import lax
from jax.experimental import pallas as pl
from jax.experimental.pallas import tpu as pltpu
from jax.experimental.pallas import tpu_sc as plsc

B = 16384
D_FRAME = 128
D_VIS = 64
D_EMB = 64
N_ACTIONS = 18
VOCAB_HALF = 500000

_NC = 2
_NS = 16
_NW = _NC * _NS
_BPW = B // _NW
_IC = _BPW // 128


@functools.cache
def _make_sc_gather():
    @functools.partial(
        pl.kernel,
        mesh=plsc.VectorSubcoreMesh(core_axis_name="c", subcore_axis_name="s"),
        out_type=jax.ShapeDtypeStruct((B, 128), jnp.float32),
        scratch_types=[
            pltpu.VMEM((_IC, 128), jnp.int32),
            pltpu.VMEM((_IC, 128), jnp.int32),
            pltpu.VMEM((_BPW, 128), jnp.float32),
            pltpu.SemaphoreType.DMA,
        ],
    )
    def _sc_gather(idx_hbm, table_hbm, out_hbm, idx_v, idx2_v, rows_v, sem):
        wid = lax.axis_index("s") * _NC + lax.axis_index("c")
        base = wid * _IC
        pltpu.sync_copy(idx_hbm.at[pl.ds(base, _IC)], idx_v)
        for j in range(_IC):
            for k in range(8):
                sl = pl.ds(k * 16, 16)
                idx2_v[j, sl] = lax.shift_right_logical(idx_v[j, sl], 1)
        copies = [
            pltpu.async_copy(table_hbm.at[idx2_v.at[j]],
                             rows_v.at[pl.ds(j * 128, 128)], sem)
            for j in range(_IC)
        ]
        for c in copies:
            c.wait()
        pltpu.sync_copy(rows_v, out_hbm.at[pl.ds(wid * _BPW, _BPW)])

    return _sc_gather


def _dense_body(frames_ref, g_ref, par_ref, wvis_ref, bvis_ref, wpol_ref,
                bpol_ref, out_ref):
    vis = jnp.dot(frames_ref[...], wvis_ref[...],
                  preferred_element_type=jnp.float32) + bvis_ref[...]
    g = g_ref[...]
    odd = (par_ref[...] & 1) == 1
    emb = jnp.where(odd, g[:, D_EMB:], g[:, :D_EMB])
    wp = wpol_ref[...]
    out_ref[...] = (
        jnp.dot(vis, wp[:D_VIS, :], preferred_element_type=jnp.float32)
        + jnp.dot(emb, wp[D_VIS:, :], preferred_element_type=jnp.float32)
        + bpol_ref[...]
    )


_BLK = 2048


def _dense(frames, g, par, W_vis, b_vis2, W_pol, b_pol2):
    return pl.pallas_call(
        _dense_body,
        grid=(B // _BLK,),
        in_specs=[
            pl.BlockSpec((_BLK, D_FRAME), lambda i: (i, 0)),
            pl.BlockSpec((_BLK, 128), lambda i: (i, 0)),
            pl.BlockSpec((_BLK, 1), lambda i: (i, 0)),
            pl.BlockSpec((D_FRAME, D_VIS), lambda i: (0, 0)),
            pl.BlockSpec((1, D_VIS), lambda i: (0, 0)),
            pl.BlockSpec((D_FRAME, N_ACTIONS), lambda i: (0, 0)),
            pl.BlockSpec((1, N_ACTIONS), lambda i: (0, 0)),
        ],
        out_specs=pl.BlockSpec((_BLK, N_ACTIONS), lambda i: (i, 0)),
        out_shape=jax.ShapeDtypeStruct((B, N_ACTIONS), jnp.float32),
    )(frames, g, par, W_vis, b_vis2, W_pol, b_pol2)


def kernel(frames, object_index, W_vis, b_vis, emb_table, W_pol, b_pol):
    idx = object_index.astype(jnp.int32)
    idx2d = idx.reshape(B // 128, 128)
    table2 = emb_table.reshape(VOCAB_HALF, 128)
    g = _make_sc_gather()(idx2d, table2)
    return _dense(frames, g, idx.reshape(B, 1), W_vis,
                  b_vis.reshape(1, D_VIS), W_pol, b_pol.reshape(1, N_ACTIONS))


# speedup vs baseline: 1.8656x; 1.8656x over previous
import functools

import jax
import jax.numpy as jnp
from jax import lax
from jax.experimental import pallas as pl
from jax.experimental.pallas import tpu as pltpu
from jax.experimental.pallas import tpu_sc as plsc

B = 16384
D_FRAME = 128
D_VIS = 64
D_EMB = 64
N_ACTIONS = 18
VOCAB = 1000000
P_ROWS = VOCAB // 4

_NC = 2
_NS = 16
_NW = _NC * _NS
_BPW = B // _NW
_IC = _BPW // 128

_PCHUNK = 8192


def _repack_body(tablet_ref, wb32_ref, out_ref, res_ref):
    res_ref[...] = lax.dot_general(tablet_ref[...], wb32_ref[...],
                                   (((0,), (0,)), ((), ())),
                                   preferred_element_type=jnp.float32)
    for j in range(4):
        out_ref[:, pl.ds(j * 32, 32)] = res_ref[pl.Slice(j, _PCHUNK // 4, 4), :]


def _repack(table_t, wb32):
    grid = (VOCAB + _PCHUNK - 1) // _PCHUNK
    return pl.pallas_call(
        _repack_body,
        grid=(grid,),
        in_specs=[
            pl.BlockSpec((D_EMB, _PCHUNK), lambda i: (0, i)),
            pl.BlockSpec((D_EMB, 32), lambda i: (0, 0)),
        ],
        out_specs=pl.BlockSpec((_PCHUNK // 4, 128), lambda i: (i, 0)),
        out_shape=jax.ShapeDtypeStruct((P_ROWS, 128), jnp.float32),
        scratch_shapes=[pltpu.VMEM((_PCHUNK, 32), jnp.float32)],
    )(table_t, wb32)


@functools.cache
def _make_sc_gather():
    @functools.partial(
        pl.kernel,
        mesh=plsc.VectorSubcoreMesh(core_axis_name="c", subcore_axis_name="s"),
        out_type=jax.ShapeDtypeStruct((B, 128), jnp.float32),
        scratch_types=[
            pltpu.VMEM((_IC, 128), jnp.int32),
            pltpu.VMEM((_IC, 128), jnp.int32),
            pltpu.VMEM((_BPW, 128), jnp.float32),
            pltpu.SemaphoreType.DMA,
        ],
    )
    def _sc_gather(idx_hbm, p_hbm, out_hbm, idx_v, idx2_v, rows_v, sem):
        wid = lax.axis_index("s") * _NC + lax.axis_index("c")
        base = wid * _IC
        pltpu.sync_copy(idx_hbm.at[pl.ds(base, _IC)], idx_v)
        for j in range(_IC):
            for k in range(8):
                sl = pl.ds(k * 16, 16)
                idx2_v[j, sl] = lax.shift_right_logical(idx_v[j, sl], 2)
        copies = [
            pltpu.async_copy(p_hbm.at[idx2_v.at[j]],
                             rows_v.at[pl.ds(j * 128, 128)], sem)
            for j in range(_IC)
        ]
        for c in copies:
            c.wait()
        pltpu.sync_copy(rows_v, out_hbm.at[pl.ds(wid * _BPW, _BPW)])

    return _sc_gather


def _dense_body(frames_ref, g_ref, sel_ref, wvis_ref, bvis_ref, wpol_ref,
                bpol_ref, out_ref):
    vis = jnp.dot(frames_ref[...], wvis_ref[...],
                  preferred_element_type=jnp.float32) + bvis_ref[...]
    g = g_ref[...]
    sel = sel_ref[...] & 3
    half = jnp.where(sel >= 2, g[:, 64:], g[:, :64])
    quarter = jnp.where((sel & 1) == 1, half[:, 32:], half[:, :32])
    out_ref[...] = (
        jnp.dot(vis, wpol_ref[:D_VIS, :], preferred_element_type=jnp.float32)
        + quarter[:, :N_ACTIONS]
        + bpol_ref[...]
    )


_BLK = 2048


def _dense(frames, g, sel, W_vis, b_vis2, W_pol, b_pol2):
    return pl.pallas_call(
        _dense_body,
        grid=(B // _BLK,),
        in_specs=[
            pl.BlockSpec((_BLK, D_FRAME), lambda i: (i, 0)),
            pl.BlockSpec((_BLK, 128), lambda i: (i, 0)),
            pl.BlockSpec((_BLK, 1), lambda i: (i, 0)),
            pl.BlockSpec((D_FRAME, D_VIS), lambda i: (0, 0)),
            pl.BlockSpec((1, D_VIS), lambda i: (0, 0)),
            pl.BlockSpec((D_FRAME, N_ACTIONS), lambda i: (0, 0)),
            pl.BlockSpec((1, N_ACTIONS), lambda i: (0, 0)),
        ],
        out_specs=pl.BlockSpec((_BLK, N_ACTIONS), lambda i: (i, 0)),
        out_shape=jax.ShapeDtypeStruct((B, N_ACTIONS), jnp.float32),
    )(frames, g, sel, W_vis, b_vis2, W_pol, b_pol2)


def kernel(frames, object_index, W_vis, b_vis, emb_table, W_pol, b_pol):
    idx = object_index.astype(jnp.int32)
    table_t = emb_table.T
    wb32 = jnp.pad(W_pol[D_VIS:, :], ((0, 0), (0, 32 - N_ACTIONS)))
    p2 = _repack(table_t, wb32)
    g = _make_sc_gather()(idx.reshape(B // 128, 128), p2)
    return _dense(frames, g, idx.reshape(B, 1), W_vis,
                  b_vis.reshape(1, D_VIS), W_pol, b_pol.reshape(1, N_ACTIONS))


# speedup vs baseline: 2.8533x; 1.5295x over previous
import functools

import jax
import jax.numpy as jnp
from jax import lax
from jax.experimental import pallas as pl
from jax.experimental.pallas import tpu as pltpu
from jax.experimental.pallas import tpu_sc as plsc

B = 16384
D_FRAME = 128
D_VIS = 64
D_EMB = 64
N_ACTIONS = 18
VOCAB = 1000000
REGION = 262144
P_ROWS = REGION

_NC = 2
_NS = 16
_NW = _NC * _NS
_BPW = B // _NW
_IC = _BPW // 128

_QBLK = 2048
_LAST_LHS_BLK = (VOCAB - 1) // _QBLK


def _repack_body(t0_ref, t1_ref, t2_ref, t3_ref, wbd_ref, out_ref):
    t4 = jnp.concatenate(
        [t0_ref[...], t1_ref[...], t2_ref[...], t3_ref[...]], axis=0)
    out_ref[...] = lax.dot_general(
        t4, wbd_ref[...], (((0,), (0,)), ((), ())),
        preferred_element_type=jnp.float32)


def _repack(table_t, wbd):
    def lhs_spec(j):
        return pl.BlockSpec(
            (D_EMB, _QBLK),
            lambda i, j=j: (0, jnp.minimum(i + (REGION // _QBLK) * j,
                                           _LAST_LHS_BLK)))

    return pl.pallas_call(
        _repack_body,
        grid=(REGION // _QBLK,),
        in_specs=[
            lhs_spec(0), lhs_spec(1), lhs_spec(2), lhs_spec(3),
            pl.BlockSpec((4 * D_EMB, 128), lambda i: (0, 0)),
        ],
        out_specs=pl.BlockSpec((_QBLK, 128), lambda i: (i, 0)),
        out_shape=jax.ShapeDtypeStruct((P_ROWS, 128), jnp.float32),
    )(table_t, table_t, table_t, table_t, wbd)


@functools.cache
def _make_sc_gather():
    @functools.partial(
        pl.kernel,
        mesh=plsc.VectorSubcoreMesh(core_axis_name="c", subcore_axis_name="s"),
        out_type=jax.ShapeDtypeStruct((B, 128), jnp.float32),
        scratch_types=[
            pltpu.VMEM((_IC, 128), jnp.int32),
            pltpu.VMEM((_IC, 128), jnp.int32),
            pltpu.VMEM((_BPW, 128), jnp.float32),
            pltpu.SemaphoreType.DMA,
        ],
    )
    def _sc_gather(idx_hbm, p_hbm, out_hbm, idx_v, idx2_v, rows_v, sem):
        wid = lax.axis_index("s") * _NC + lax.axis_index("c")
        base = wid * _IC
        pltpu.sync_copy(idx_hbm.at[pl.ds(base, _IC)], idx_v)
        for j in range(_IC):
            for k in range(8):
                sl = pl.ds(k * 16, 16)
                idx2_v[j, sl] = idx_v[j, sl] & (REGION - 1)
        copies = [
            pltpu.async_copy(p_hbm.at[idx2_v.at[j]],
                             rows_v.at[pl.ds(j * 128, 128)], sem)
            for j in range(_IC)
        ]
        for c in copies:
            c.wait()
        pltpu.sync_copy(rows_v, out_hbm.at[pl.ds(wid * _BPW, _BPW)])

    return _sc_gather


def _dense_body(frames_ref, g_ref, sel_ref, wvis_ref, bvis_ref, wpol_ref,
                bpol_ref, out_ref):
    vis = jnp.dot(frames_ref[...], wvis_ref[...],
                  preferred_element_type=jnp.float32) + bvis_ref[...]
    g = g_ref[...]
    j = sel_ref[...] >> 18
    half = jnp.where(j >= 2, g[:, 64:], g[:, :64])
    quarter = jnp.where((j & 1) == 1, half[:, 32:], half[:, :32])
    out_ref[...] = (
        jnp.dot(vis, wpol_ref[:D_VIS, :], preferred_element_type=jnp.float32)
        + quarter[:, :N_ACTIONS]
        + bpol_ref[...]
    )


_BLK = 2048


def _dense(frames, g, sel, W_vis, b_vis2, W_pol, b_pol2):
    return pl.pallas_call(
        _dense_body,
        grid=(B // _BLK,),
        in_specs=[
            pl.BlockSpec((_BLK, D_FRAME), lambda i: (i, 0)),
            pl.BlockSpec((_BLK, 128), lambda i: (i, 0)),
            pl.BlockSpec((_BLK, 1), lambda i: (i, 0)),
            pl.BlockSpec((D_FRAME, D_VIS), lambda i: (0, 0)),
            pl.BlockSpec((1, D_VIS), lambda i: (0, 0)),
            pl.BlockSpec((D_FRAME, N_ACTIONS), lambda i: (0, 0)),
            pl.BlockSpec((1, N_ACTIONS), lambda i: (0, 0)),
        ],
        out_specs=pl.BlockSpec((_BLK, N_ACTIONS), lambda i: (i, 0)),
        out_shape=jax.ShapeDtypeStruct((B, N_ACTIONS), jnp.float32),
    )(frames, g, sel, W_vis, b_vis2, W_pol, b_pol2)


def kernel(frames, object_index, W_vis, b_vis, emb_table, W_pol, b_pol):
    idx = object_index.astype(jnp.int32)
    table_t = emb_table.T
    wb32 = jnp.pad(W_pol[D_VIS:, :], ((0, 0), (0, 32 - N_ACTIONS)))
    wbd = jax.scipy.linalg.block_diag(wb32, wb32, wb32, wb32)
    p2 = _repack(table_t, wbd)
    g = _make_sc_gather()(idx.reshape(B // 128, 128), p2)
    return _dense(frames, g, idx.reshape(B, 1), W_vis,
                  b_vis.reshape(1, D_VIS), W_pol, b_pol.reshape(1, N_ACTIONS))


# speedup vs baseline: 3.4848x; 1.2213x over previous
import functools

import jax
import jax.numpy as jnp
from jax import lax
from jax.experimental import pallas as pl
from jax.experimental.pallas import tpu as pltpu
from jax.experimental.pallas import tpu_sc as plsc

B = 16384
D_FRAME = 128
D_VIS = 64
D_EMB = 64
N_ACTIONS = 18
VOCAB = 1000000
REGION = 262144
P_ROWS = REGION

_NC = 2
_NS = 16
_NW = _NC * _NS
_BPW = B // _NW
_IC = _BPW // 128

_QBLK = 4096
_LAST_LHS_BLK = (VOCAB - 1) // _QBLK


def _repack_body(t0_ref, t1_ref, t2_ref, t3_ref, wbd_ref, out_ref):
    t4 = jnp.concatenate(
        [t0_ref[...], t1_ref[...], t2_ref[...], t3_ref[...]], axis=0)
    out_ref[...] = lax.dot_general(
        t4, wbd_ref[...], (((0,), (0,)), ((), ())),
        preferred_element_type=jnp.float32)


def _repack(table_t, wbd):
    def lhs_spec(j):
        return pl.BlockSpec(
            (D_EMB, _QBLK),
            lambda i, j=j: (0, jnp.minimum(i + (REGION // _QBLK) * j,
                                           _LAST_LHS_BLK)))

    return pl.pallas_call(
        _repack_body,
        grid=(REGION // _QBLK,),
        in_specs=[
            lhs_spec(0), lhs_spec(1), lhs_spec(2), lhs_spec(3),
            pl.BlockSpec((4 * D_EMB, 128), lambda i: (0, 0)),
        ],
        out_specs=pl.BlockSpec((_QBLK, 128), lambda i: (i, 0)),
        out_shape=jax.ShapeDtypeStruct((P_ROWS, 128), jnp.float32),
    )(table_t, table_t, table_t, table_t, wbd)


@functools.cache
def _make_sc_gather():
    @functools.partial(
        pl.kernel,
        mesh=plsc.VectorSubcoreMesh(core_axis_name="c", subcore_axis_name="s"),
        out_type=jax.ShapeDtypeStruct((B, 128), jnp.float32),
        scratch_types=[
            pltpu.VMEM((_IC, 128), jnp.int32),
            pltpu.VMEM((_IC, 128), jnp.int32),
            pltpu.VMEM((_BPW, 128), jnp.float32),
            pltpu.SemaphoreType.DMA,
        ],
    )
    def _sc_gather(idx_hbm, p_hbm, out_hbm, idx_v, idx2_v, rows_v, sem):
        wid = lax.axis_index("s") * _NC + lax.axis_index("c")
        base = wid * _IC
        pltpu.sync_copy(idx_hbm.at[pl.ds(base, _IC)], idx_v)
        for j in range(_IC):
            for k in range(8):
                sl = pl.ds(k * 16, 16)
                idx2_v[j, sl] = idx_v[j, sl] & (REGION - 1)
        copies = [
            pltpu.async_copy(p_hbm.at[idx2_v.at[j]],
                             rows_v.at[pl.ds(j * 128, 128)], sem)
            for j in range(_IC)
        ]
        for c in copies:
            c.wait()
        pltpu.sync_copy(rows_v, out_hbm.at[pl.ds(wid * _BPW, _BPW)])

    return _sc_gather


def _dense_body(frames_ref, g_ref, sel_ref, wvis_ref, bvis_ref, wpol_ref,
                bpol_ref, out_ref):
    vis = jnp.dot(frames_ref[...], wvis_ref[...],
                  preferred_element_type=jnp.float32) + bvis_ref[...]
    g = g_ref[...]
    j = sel_ref[...] >> 18
    half = jnp.where(j >= 2, g[:, 64:], g[:, :64])
    quarter = jnp.where((j & 1) == 1, half[:, 32:], half[:, :32])
    out_ref[...] = (
        jnp.dot(vis, wpol_ref[:D_VIS, :], preferred_element_type=jnp.float32)
        + quarter[:, :N_ACTIONS]
        + bpol_ref[...]
    )


_BLK = 2048


def _dense(frames, g, sel, W_vis, b_vis2, W_pol, b_pol2):
    return pl.pallas_call(
        _dense_body,
        grid=(B // _BLK,),
        in_specs=[
            pl.BlockSpec((_BLK, D_FRAME), lambda i: (i, 0)),
            pl.BlockSpec((_BLK, 128), lambda i: (i, 0)),
            pl.BlockSpec((_BLK, 1), lambda i: (i, 0)),
            pl.BlockSpec((D_FRAME, D_VIS), lambda i: (0, 0)),
            pl.BlockSpec((1, D_VIS), lambda i: (0, 0)),
            pl.BlockSpec((D_FRAME, N_ACTIONS), lambda i: (0, 0)),
            pl.BlockSpec((1, N_ACTIONS), lambda i: (0, 0)),
        ],
        out_specs=pl.BlockSpec((_BLK, N_ACTIONS), lambda i: (i, 0)),
        out_shape=jax.ShapeDtypeStruct((B, N_ACTIONS), jnp.float32),
    )(frames, g, sel, W_vis, b_vis2, W_pol, b_pol2)


def kernel(frames, object_index, W_vis, b_vis, emb_table, W_pol, b_pol):
    idx = object_index.astype(jnp.int32)
    table_t = emb_table.T
    wb32 = jnp.pad(W_pol[D_VIS:, :], ((0, 0), (0, 32 - N_ACTIONS)))
    wbd = jax.scipy.linalg.block_diag(wb32, wb32, wb32, wb32)
    p2 = _repack(table_t, wbd)
    g = _make_sc_gather()(idx.reshape(B // 128, 128), p2)
    return _dense(frames, g, idx.reshape(B, 1), W_vis,
                  b_vis.reshape(1, D_VIS), W_pol, b_pol.reshape(1, N_ACTIONS))


# speedup vs baseline: 3.6677x; 1.0525x over previous
import functools

import jax
import jax.numpy as jnp
from jax import lax
from jax.experimental import pallas as pl
from jax.experimental.pallas import tpu as pltpu
from jax.experimental.pallas import tpu_sc as plsc

B = 16384
D_FRAME = 128
D_VIS = 64
D_EMB = 64
N_ACTIONS = 18
VOCAB = 1000000
REGION = 262144
P_ROWS = REGION

_NC = 2
_NS = 16
_NW = _NC * _NS
_BPW = B // _NW
_IC = _BPW // 128

_QBLK = 8192
_LAST_LHS_BLK = (VOCAB - 1) // _QBLK


def _repack_body(t0_ref, t1_ref, t2_ref, t3_ref, wbd_ref, out_ref):
    t4 = jnp.concatenate(
        [t0_ref[...], t1_ref[...], t2_ref[...], t3_ref[...]], axis=0)
    out_ref[...] = lax.dot_general(
        t4, wbd_ref[...], (((0,), (0,)), ((), ())),
        preferred_element_type=jnp.float32)


def _repack(table_t, wbd):
    def lhs_spec(j):
        return pl.BlockSpec(
            (D_EMB, _QBLK),
            lambda i, j=j: (0, jnp.minimum(i + (REGION // _QBLK) * j,
                                           _LAST_LHS_BLK)))

    return pl.pallas_call(
        _repack_body,
        grid=(REGION // _QBLK,),
        in_specs=[
            lhs_spec(0), lhs_spec(1), lhs_spec(2), lhs_spec(3),
            pl.BlockSpec((4 * D_EMB, 128), lambda i: (0, 0)),
        ],
        out_specs=pl.BlockSpec((_QBLK, 128), lambda i: (i, 0)),
        out_shape=jax.ShapeDtypeStruct((P_ROWS, 128), jnp.float32),
    )(table_t, table_t, table_t, table_t, wbd)


@functools.cache
def _make_sc_gather():
    @functools.partial(
        pl.kernel,
        mesh=plsc.VectorSubcoreMesh(core_axis_name="c", subcore_axis_name="s"),
        out_type=jax.ShapeDtypeStruct((B, 128), jnp.float32),
        scratch_types=[
            pltpu.VMEM((_IC, 128), jnp.int32),
            pltpu.VMEM((_IC, 128), jnp.int32),
            pltpu.VMEM((_BPW, 128), jnp.float32),
            pltpu.SemaphoreType.DMA,
        ],
    )
    def _sc_gather(idx_hbm, p_hbm, out_hbm, idx_v, idx2_v, rows_v, sem):
        wid = lax.axis_index("s") * _NC + lax.axis_index("c")
        base = wid * _IC
        pltpu.sync_copy(idx_hbm.at[pl.ds(base, _IC)], idx_v)
        for j in range(_IC):
            for k in range(8):
                sl = pl.ds(k * 16, 16)
                idx2_v[j, sl] = idx_v[j, sl] & (REGION - 1)
        copies = [
            pltpu.async_copy(p_hbm.at[idx2_v.at[j]],
                             rows_v.at[pl.ds(j * 128, 128)], sem)
            for j in range(_IC)
        ]
        for c in copies:
            c.wait()
        pltpu.sync_copy(rows_v, out_hbm.at[pl.ds(wid * _BPW, _BPW)])

    return _sc_gather


def _dense_body(frames_ref, g_ref, sel_ref, wvis_ref, bvis_ref, wpol_ref,
                bpol_ref, out_ref):
    vis = jnp.dot(frames_ref[...], wvis_ref[...],
                  preferred_element_type=jnp.float32) + bvis_ref[...]
    g = g_ref[...]
    j = sel_ref[...] >> 18
    half = jnp.where(j >= 2, g[:, 64:], g[:, :64])
    quarter = jnp.where((j & 1) == 1, half[:, 32:], half[:, :32])
    out_ref[...] = (
        jnp.dot(vis, wpol_ref[:D_VIS, :], preferred_element_type=jnp.float32)
        + quarter[:, :N_ACTIONS]
        + bpol_ref[...]
    )


_BLK = 2048


def _dense(frames, g, sel, W_vis, b_vis2, W_pol, b_pol2):
    return pl.pallas_call(
        _dense_body,
        grid=(B // _BLK,),
        in_specs=[
            pl.BlockSpec((_BLK, D_FRAME), lambda i: (i, 0)),
            pl.BlockSpec((_BLK, 128), lambda i: (i, 0)),
            pl.BlockSpec((_BLK, 1), lambda i: (i, 0)),
            pl.BlockSpec((D_FRAME, D_VIS), lambda i: (0, 0)),
            pl.BlockSpec((1, D_VIS), lambda i: (0, 0)),
            pl.BlockSpec((D_FRAME, N_ACTIONS), lambda i: (0, 0)),
            pl.BlockSpec((1, N_ACTIONS), lambda i: (0, 0)),
        ],
        out_specs=pl.BlockSpec((_BLK, N_ACTIONS), lambda i: (i, 0)),
        out_shape=jax.ShapeDtypeStruct((B, N_ACTIONS), jnp.float32),
    )(frames, g, sel, W_vis, b_vis2, W_pol, b_pol2)


def kernel(frames, object_index, W_vis, b_vis, emb_table, W_pol, b_pol):
    idx = object_index.astype(jnp.int32)
    table_t = emb_table.T
    wb32 = jnp.pad(W_pol[D_VIS:, :], ((0, 0), (0, 32 - N_ACTIONS)))
    wbd = jax.scipy.linalg.block_diag(wb32, wb32, wb32, wb32)
    p2 = _repack(table_t, wbd)
    g = _make_sc_gather()(idx.reshape(B // 128, 128), p2)
    return _dense(frames, g, idx.reshape(B, 1), W_vis,
                  b_vis.reshape(1, D_VIS), W_pol, b_pol.reshape(1, N_ACTIONS))


# speedup vs baseline: 3.7482x; 1.0220x over previous
import functools

import jax
import jax.numpy as jnp
from jax import lax
from jax.experimental import pallas as pl
from jax.experimental.pallas import tpu as pltpu
from jax.experimental.pallas import tpu_sc as plsc

B = 16384
D_FRAME = 128
D_VIS = 64
D_EMB = 64
N_ACTIONS = 18
VOCAB = 1000000
REGION = 262144
P_ROWS = REGION

_NC = 2
_NS = 16
_NW = _NC * _NS
_BPW = B // _NW
_IC = _BPW // 128

_QBLK = 16384
_LAST_LHS_BLK = (VOCAB - 1) // _QBLK


def _repack_body(t0_ref, t1_ref, t2_ref, t3_ref, wbd_ref, out_ref):
    t4 = jnp.concatenate(
        [t0_ref[...], t1_ref[...], t2_ref[...], t3_ref[...]], axis=0)
    out_ref[...] = lax.dot_general(
        t4, wbd_ref[...], (((0,), (0,)), ((), ())),
        preferred_element_type=jnp.float32)


def _repack(table_t, wbd):
    def lhs_spec(j):
        return pl.BlockSpec(
            (D_EMB, _QBLK),
            lambda i, j=j: (0, jnp.minimum(i + (REGION // _QBLK) * j,
                                           _LAST_LHS_BLK)))

    return pl.pallas_call(
        _repack_body,
        grid=(REGION // _QBLK,),
        in_specs=[
            lhs_spec(0), lhs_spec(1), lhs_spec(2), lhs_spec(3),
            pl.BlockSpec((4 * D_EMB, 128), lambda i: (0, 0)),
        ],
        out_specs=pl.BlockSpec((_QBLK, 128), lambda i: (i, 0)),
        out_shape=jax.ShapeDtypeStruct((P_ROWS, 128), jnp.float32),
    )(table_t, table_t, table_t, table_t, wbd)


@functools.cache
def _make_sc_gather():
    @functools.partial(
        pl.kernel,
        mesh=plsc.VectorSubcoreMesh(core_axis_name="c", subcore_axis_name="s"),
        out_type=jax.ShapeDtypeStruct((B, 128), jnp.float32),
        scratch_types=[
            pltpu.VMEM((_IC, 128), jnp.int32),
            pltpu.VMEM((_IC, 128), jnp.int32),
            pltpu.VMEM((_BPW, 128), jnp.float32),
            pltpu.SemaphoreType.DMA,
        ],
    )
    def _sc_gather(idx_hbm, p_hbm, out_hbm, idx_v, idx2_v, rows_v, sem):
        wid = lax.axis_index("s") * _NC + lax.axis_index("c")
        base = wid * _IC
        pltpu.sync_copy(idx_hbm.at[pl.ds(base, _IC)], idx_v)
        for j in range(_IC):
            for k in range(8):
                sl = pl.ds(k * 16, 16)
                idx2_v[j, sl] = idx_v[j, sl] & (REGION - 1)
        copies = [
            pltpu.async_copy(p_hbm.at[idx2_v.at[j]],
                             rows_v.at[pl.ds(j * 128, 128)], sem)
            for j in range(_IC)
        ]
        for c in copies:
            c.wait()
        pltpu.sync_copy(rows_v, out_hbm.at[pl.ds(wid * _BPW, _BPW)])

    return _sc_gather


def _dense_body(frames_ref, g_ref, sel_ref, wvis_ref, bvis_ref, wpol_ref,
                bpol_ref, out_ref):
    vis = jnp.dot(frames_ref[...], wvis_ref[...],
                  preferred_element_type=jnp.float32) + bvis_ref[...]
    g = g_ref[...]
    j = sel_ref[...] >> 18
    half = jnp.where(j >= 2, g[:, 64:], g[:, :64])
    quarter = jnp.where((j & 1) == 1, half[:, 32:], half[:, :32])
    out_ref[...] = (
        jnp.dot(vis, wpol_ref[:D_VIS, :], preferred_element_type=jnp.float32)
        + quarter[:, :N_ACTIONS]
        + bpol_ref[...]
    )


_BLK = 2048


def _dense(frames, g, sel, W_vis, b_vis2, W_pol, b_pol2):
    return pl.pallas_call(
        _dense_body,
        grid=(B // _BLK,),
        in_specs=[
            pl.BlockSpec((_BLK, D_FRAME), lambda i: (i, 0)),
            pl.BlockSpec((_BLK, 128), lambda i: (i, 0)),
            pl.BlockSpec((_BLK, 1), lambda i: (i, 0)),
            pl.BlockSpec((D_FRAME, D_VIS), lambda i: (0, 0)),
            pl.BlockSpec((1, D_VIS), lambda i: (0, 0)),
            pl.BlockSpec((D_FRAME, N_ACTIONS), lambda i: (0, 0)),
            pl.BlockSpec((1, N_ACTIONS), lambda i: (0, 0)),
        ],
        out_specs=pl.BlockSpec((_BLK, N_ACTIONS), lambda i: (i, 0)),
        out_shape=jax.ShapeDtypeStruct((B, N_ACTIONS), jnp.float32),
    )(frames, g, sel, W_vis, b_vis2, W_pol, b_pol2)


def kernel(frames, object_index, W_vis, b_vis, emb_table, W_pol, b_pol):
    idx = object_index.astype(jnp.int32)
    table_t = emb_table.T
    wb32 = jnp.pad(W_pol[D_VIS:, :], ((0, 0), (0, 32 - N_ACTIONS)))
    wbd = jax.scipy.linalg.block_diag(wb32, wb32, wb32, wb32)
    p2 = _repack(table_t, wbd)
    g = _make_sc_gather()(idx.reshape(B // 128, 128), p2)
    return _dense(frames, g, idx.reshape(B, 1), W_vis,
                  b_vis.reshape(1, D_VIS), W_pol, b_pol.reshape(1, N_ACTIONS))


# speedup vs baseline: 4.1459x; 1.1061x over previous
import functools

import jax
import jax.numpy as jnp
from jax import lax
from jax.experimental import pallas as pl
from jax.experimental.pallas import tpu as pltpu
from jax.experimental.pallas import tpu_sc as plsc

B = 16384
D_FRAME = 128
D_VIS = 64
D_EMB = 64
N_ACTIONS = 18
VOCAB = 1000000
REGION = 131072
P_ROWS = REGION

_NC = 2
_NS = 16
_NW = _NC * _NS
_BPW = B // _NW
_IC = _BPW // 128

_QBLK = 4096
_LAST_LHS_BLK = (VOCAB - 1) // _QBLK


def _bf16_bits(x):
    b = lax.bitcast_convert_type(x.astype(jnp.bfloat16), jnp.uint16)
    return b.astype(jnp.uint32)


def _repack_body(t0_ref, t1_ref, t2_ref, t3_ref, t4_ref, t5_ref, t6_ref,
                 t7_ref, wbd_ref, out_ref):
    wbd = wbd_ref[...]
    t_hi = jnp.concatenate(
        [t0_ref[...], t2_ref[...], t4_ref[...], t6_ref[...]], axis=0)
    t_lo = jnp.concatenate(
        [t1_ref[...], t3_ref[...], t5_ref[...], t7_ref[...]], axis=0)
    res_hi = lax.dot_general(t_hi, wbd, (((0,), (0,)), ((), ())),
                             preferred_element_type=jnp.float32)
    res_lo = lax.dot_general(t_lo, wbd, (((0,), (0,)), ((), ())),
                             preferred_element_type=jnp.float32)
    packed = (_bf16_bits(res_hi) << 16) | _bf16_bits(res_lo)
    out_ref[...] = lax.bitcast_convert_type(packed, jnp.float32)


def _repack(table_t, wbd):
    def lhs_spec(j):
        return pl.BlockSpec(
            (D_EMB, _QBLK),
            lambda i, j=j: (0, jnp.minimum(i + (REGION // _QBLK) * j,
                                           _LAST_LHS_BLK)))

    return pl.pallas_call(
        _repack_body,
        grid=(REGION // _QBLK,),
        in_specs=[
            lhs_spec(0), lhs_spec(1), lhs_spec(2), lhs_spec(3),
            lhs_spec(4), lhs_spec(5), lhs_spec(6), lhs_spec(7),
            pl.BlockSpec((4 * D_EMB, 128), lambda i: (0, 0)),
        ],
        out_specs=pl.BlockSpec((_QBLK, 128), lambda i: (i, 0)),
        out_shape=jax.ShapeDtypeStruct((P_ROWS, 128), jnp.float32),
    )(table_t, table_t, table_t, table_t, table_t, table_t, table_t, table_t,
      wbd)


@functools.cache
def _make_sc_gather():
    @functools.partial(
        pl.kernel,
        mesh=plsc.VectorSubcoreMesh(core_axis_name="c", subcore_axis_name="s"),
        out_type=jax.ShapeDtypeStruct((B, 128), jnp.float32),
        scratch_types=[
            pltpu.VMEM((_IC, 128), jnp.int32),
            pltpu.VMEM((_IC, 128), jnp.int32),
            pltpu.VMEM((_BPW, 128), jnp.float32),
            pltpu.SemaphoreType.DMA,
        ],
    )
    def _sc_gather(idx_hbm, p_hbm, out_hbm, idx_v, idx2_v, rows_v, sem):
        wid = lax.axis_index("s") * _NC + lax.axis_index("c")
        base = wid * _IC
        pltpu.sync_copy(idx_hbm.at[pl.ds(base, _IC)], idx_v)
        for j in range(_IC):
            for k in range(8):
                sl = pl.ds(k * 16, 16)
                idx2_v[j, sl] = idx_v[j, sl] & (REGION - 1)
        copies = [
            pltpu.async_copy(p_hbm.at[idx2_v.at[j]],
                             rows_v.at[pl.ds(j * 128, 128)], sem)
            for j in range(_IC)
        ]
        for c in copies:
            c.wait()
        pltpu.sync_copy(rows_v, out_hbm.at[pl.ds(wid * _BPW, _BPW)])

    return _sc_gather


def _dense_body(frames_ref, g_ref, sel_ref, wvis_ref, bvis_ref, wpol_ref,
                bpol_ref, out_ref):
    vis = jnp.dot(frames_ref[...], wvis_ref[...],
                  preferred_element_type=jnp.float32) + bvis_ref[...]
    g = g_ref[...]
    j = sel_ref[...] >> 17
    k = j >> 1
    half = jnp.where(k >= 2, g[:, 64:], g[:, :64])
    quarter = jnp.where((k & 1) == 1, half[:, 32:], half[:, :32])
    u = lax.bitcast_convert_type(quarter, jnp.uint32)
    val = jnp.where((j & 1) == 0, u & jnp.uint32(0xFFFF0000), u << 16)
    emb = lax.bitcast_convert_type(val, jnp.float32)
    out_ref[...] = (
        jnp.dot(vis, wpol_ref[:D_VIS, :], preferred_element_type=jnp.float32)
        + emb[:, :N_ACTIONS]
        + bpol_ref[...]
    )


_BLK = 2048


def _dense(frames, g, sel, W_vis, b_vis2, W_pol, b_pol2):
    return pl.pallas_call(
        _dense_body,
        grid=(B // _BLK,),
        in_specs=[
            pl.BlockSpec((_BLK, D_FRAME), lambda i: (i, 0)),
            pl.BlockSpec((_BLK, 128), lambda i: (i, 0)),
            pl.BlockSpec((_BLK, 1), lambda i: (i, 0)),
            pl.BlockSpec((D_FRAME, D_VIS), lambda i: (0, 0)),
            pl.BlockSpec((1, D_VIS), lambda i: (0, 0)),
            pl.BlockSpec((D_FRAME, N_ACTIONS), lambda i: (0, 0)),
            pl.BlockSpec((1, N_ACTIONS), lambda i: (0, 0)),
        ],
        out_specs=pl.BlockSpec((_BLK, N_ACTIONS), lambda i: (i, 0)),
        out_shape=jax.ShapeDtypeStruct((B, N_ACTIONS), jnp.float32),
    )(frames, g, sel, W_vis, b_vis2, W_pol, b_pol2)


def kernel(frames, object_index, W_vis, b_vis, emb_table, W_pol, b_pol):
    idx = object_index.astype(jnp.int32)
    table_t = emb_table.T
    wb32 = jnp.pad(W_pol[D_VIS:, :], ((0, 0), (0, 32 - N_ACTIONS)))
    wbd = jax.scipy.linalg.block_diag(wb32, wb32, wb32, wb32)
    p2 = _repack(table_t, wbd)
    g = _make_sc_gather()(idx.reshape(B // 128, 128), p2)
    return _dense(frames, g, idx.reshape(B, 1), W_vis,
                  b_vis.reshape(1, D_VIS), W_pol, b_pol.reshape(1, N_ACTIONS))


# speedup vs baseline: 4.1891x; 1.0104x over previous
import functools

import jax
import jax.numpy as jnp
from jax import lax
from jax.experimental import pallas as pl
from jax.experimental.pallas import tpu as pltpu
from jax.experimental.pallas import tpu_sc as plsc

B = 16384
D_FRAME = 128
D_VIS = 64
D_EMB = 64
N_ACTIONS = 18
VOCAB = 1000000
REGION = 131072
P_ROWS = REGION

_NC = 2
_NS = 16
_NW = _NC * _NS
_BPW = B // _NW
_IC = _BPW // 128

_QBLK = 8192
_LAST_LHS_BLK = (VOCAB - 1) // _QBLK


def _bf16_bits(x):
    b = lax.bitcast_convert_type(x.astype(jnp.bfloat16), jnp.uint16)
    return b.astype(jnp.uint32)


def _repack_body(t0_ref, t1_ref, t2_ref, t3_ref, t4_ref, t5_ref, t6_ref,
                 t7_ref, wbd_ref, out_ref):
    wbd = wbd_ref[...]
    t_hi = jnp.concatenate(
        [t0_ref[...], t2_ref[...], t4_ref[...], t6_ref[...]], axis=0)
    t_lo = jnp.concatenate(
        [t1_ref[...], t3_ref[...], t5_ref[...], t7_ref[...]], axis=0)
    res_hi = lax.dot_general(t_hi, wbd, (((0,), (0,)), ((), ())),
                             preferred_element_type=jnp.float32)
    res_lo = lax.dot_general(t_lo, wbd, (((0,), (0,)), ((), ())),
                             preferred_element_type=jnp.float32)
    packed = (_bf16_bits(res_hi) << 16) | _bf16_bits(res_lo)
    out_ref[...] = lax.bitcast_convert_type(packed, jnp.float32)


def _repack(table_t, wbd):
    def lhs_spec(j):
        return pl.BlockSpec(
            (D_EMB, _QBLK),
            lambda i, j=j: (0, jnp.minimum(i + (REGION // _QBLK) * j,
                                           _LAST_LHS_BLK)))

    return pl.pallas_call(
        _repack_body,
        grid=(REGION // _QBLK,),
        in_specs=[
            lhs_spec(0), lhs_spec(1), lhs_spec(2), lhs_spec(3),
            lhs_spec(4), lhs_spec(5), lhs_spec(6), lhs_spec(7),
            pl.BlockSpec((4 * D_EMB, 128), lambda i: (0, 0)),
        ],
        out_specs=pl.BlockSpec((_QBLK, 128), lambda i: (i, 0)),
        out_shape=jax.ShapeDtypeStruct((P_ROWS, 128), jnp.float32),
    )(table_t, table_t, table_t, table_t, table_t, table_t, table_t, table_t,
      wbd)


@functools.cache
def _make_sc_gather():
    @functools.partial(
        pl.kernel,
        mesh=plsc.VectorSubcoreMesh(core_axis_name="c", subcore_axis_name="s"),
        out_type=jax.ShapeDtypeStruct((B, 128), jnp.float32),
        scratch_types=[
            pltpu.VMEM((_IC, 128), jnp.int32),
            pltpu.VMEM((_IC, 128), jnp.int32),
            pltpu.VMEM((_BPW, 128), jnp.float32),
            pltpu.SemaphoreType.DMA,
        ],
    )
    def _sc_gather(idx_hbm, p_hbm, out_hbm, idx_v, idx2_v, rows_v, sem):
        wid = lax.axis_index("s") * _NC + lax.axis_index("c")
        base = wid * _IC
        pltpu.sync_copy(idx_hbm.at[pl.ds(base, _IC)], idx_v)
        for j in range(_IC):
            for k in range(8):
                sl = pl.ds(k * 16, 16)
                idx2_v[j, sl] = idx_v[j, sl] & (REGION - 1)
        copies = [
            pltpu.async_copy(p_hbm.at[idx2_v.at[j]],
                             rows_v.at[pl.ds(j * 128, 128)], sem)
            for j in range(_IC)
        ]
        for c in copies:
            c.wait()
        pltpu.sync_copy(rows_v, out_hbm.at[pl.ds(wid * _BPW, _BPW)])

    return _sc_gather


def _dense_body(frames_ref, g_ref, sel_ref, wvis_ref, bvis_ref, wpol_ref,
                bpol_ref, out_ref):
    vis = jnp.dot(frames_ref[...], wvis_ref[...],
                  preferred_element_type=jnp.float32) + bvis_ref[...]
    g = g_ref[...]
    j = sel_ref[...] >> 17
    k = j >> 1
    half = jnp.where(k >= 2, g[:, 64:], g[:, :64])
    quarter = jnp.where((k & 1) == 1, half[:, 32:], half[:, :32])
    u = lax.bitcast_convert_type(quarter, jnp.uint32)
    val = jnp.where((j & 1) == 0, u & jnp.uint32(0xFFFF0000), u << 16)
    emb = lax.bitcast_convert_type(val, jnp.float32)
    out_ref[...] = (
        jnp.dot(vis, wpol_ref[:D_VIS, :], preferred_element_type=jnp.float32)
        + emb[:, :N_ACTIONS]
        + bpol_ref[...]
    )


_BLK = 2048


def _dense(frames, g, sel, W_vis, b_vis2, W_pol, b_pol2):
    return pl.pallas_call(
        _dense_body,
        grid=(B // _BLK,),
        in_specs=[
            pl.BlockSpec((_BLK, D_FRAME), lambda i: (i, 0)),
            pl.BlockSpec((_BLK, 128), lambda i: (i, 0)),
            pl.BlockSpec((_BLK, 1), lambda i: (i, 0)),
            pl.BlockSpec((D_FRAME, D_VIS), lambda i: (0, 0)),
            pl.BlockSpec((1, D_VIS), lambda i: (0, 0)),
            pl.BlockSpec((D_FRAME, N_ACTIONS), lambda i: (0, 0)),
            pl.BlockSpec((1, N_ACTIONS), lambda i: (0, 0)),
        ],
        out_specs=pl.BlockSpec((_BLK, N_ACTIONS), lambda i: (i, 0)),
        out_shape=jax.ShapeDtypeStruct((B, N_ACTIONS), jnp.float32),
    )(frames, g, sel, W_vis, b_vis2, W_pol, b_pol2)


def kernel(frames, object_index, W_vis, b_vis, emb_table, W_pol, b_pol):
    idx = object_index.astype(jnp.int32)
    table_t = emb_table.T
    wb32 = jnp.pad(W_pol[D_VIS:, :], ((0, 0), (0, 32 - N_ACTIONS)))
    wbd = jax.scipy.linalg.block_diag(wb32, wb32, wb32, wb32)
    p2 = _repack(table_t, wbd)
    g = _make_sc_gather()(idx.reshape(B // 128, 128), p2)
    return _dense(frames, g, idx.reshape(B, 1), W_vis,
                  b_vis.reshape(1, D_VIS), W_pol, b_pol.reshape(1, N_ACTIONS))


# speedup vs baseline: 4.2125x; 1.0056x over previous
import functools

import jax
import jax.numpy as jnp
from jax import lax
from jax.experimental import pallas as pl
from jax.experimental.pallas import tpu as pltpu
from jax.experimental.pallas import tpu_sc as plsc

B = 16384
D_FRAME = 128
D_VIS = 64
D_EMB = 64
N_ACTIONS = 18
VOCAB = 1000000
REGION = 131072
P_ROWS = REGION

_NC = 2
_NS = 16
_NW = _NC * _NS
_BPW = B // _NW
_IC = _BPW // 128

_QBLK = 8192
_LAST_LHS_BLK = (VOCAB - 1) // _QBLK


def _bf16_bits(x):
    b = lax.bitcast_convert_type(x.astype(jnp.bfloat16), jnp.uint16)
    return b.astype(jnp.uint32)


def _repack_body(t0_ref, t1_ref, t2_ref, t3_ref, t4_ref, t5_ref, t6_ref,
                 t7_ref, wbd_ref, out_ref):
    wbd = wbd_ref[...]
    t_hi = jnp.concatenate(
        [t0_ref[...], t2_ref[...], t4_ref[...], t6_ref[...]], axis=0)
    t_lo = jnp.concatenate(
        [t1_ref[...], t3_ref[...], t5_ref[...], t7_ref[...]], axis=0)
    res_hi = lax.dot_general(t_hi, wbd, (((0,), (0,)), ((), ())),
                             preferred_element_type=jnp.float32)
    res_lo = lax.dot_general(t_lo, wbd, (((0,), (0,)), ((), ())),
                             preferred_element_type=jnp.float32)
    packed = (_bf16_bits(res_hi) << 16) | _bf16_bits(res_lo)
    out_ref[...] = lax.bitcast_convert_type(packed, jnp.float32)


def _repack(table_t, wbd):
    def lhs_spec(j):
        return pl.BlockSpec(
            (D_EMB, _QBLK),
            lambda i, j=j: (0, jnp.minimum(i + (REGION // _QBLK) * j,
                                           _LAST_LHS_BLK)))

    return pl.pallas_call(
        _repack_body,
        grid=(REGION // _QBLK,),
        in_specs=[
            lhs_spec(0), lhs_spec(1), lhs_spec(2), lhs_spec(3),
            lhs_spec(4), lhs_spec(5), lhs_spec(6), lhs_spec(7),
            pl.BlockSpec((4 * D_EMB, 128), lambda i: (0, 0)),
        ],
        out_specs=pl.BlockSpec((_QBLK, 128), lambda i: (i, 0)),
        out_shape=jax.ShapeDtypeStruct((P_ROWS, 128), jnp.float32),
    )(table_t, table_t, table_t, table_t, table_t, table_t, table_t, table_t,
      wbd)


@functools.cache
def _make_sc_gather():
    @functools.partial(
        pl.kernel,
        mesh=plsc.VectorSubcoreMesh(core_axis_name="c", subcore_axis_name="s"),
        out_type=jax.ShapeDtypeStruct((B, 128), jnp.float32),
        scratch_types=[
            pltpu.VMEM((_IC, 128), jnp.int32),
            pltpu.VMEM((_IC, 128), jnp.int32),
            pltpu.VMEM((_BPW, 128), jnp.float32),
            pltpu.SemaphoreType.DMA,
        ],
    )
    def _sc_gather(idx_hbm, p_hbm, out_hbm, idx_v, idx2_v, rows_v, sem):
        wid = lax.axis_index("s") * _NC + lax.axis_index("c")
        base = wid * _IC
        pltpu.sync_copy(idx_hbm.at[pl.ds(base, _IC)], idx_v)
        for j in range(_IC):
            for k in range(8):
                sl = pl.ds(k * 16, 16)
                idx2_v[j, sl] = idx_v[j, sl] & (REGION - 1)
        copies = [
            pltpu.async_copy(p_hbm.at[idx2_v.at[j]],
                             rows_v.at[pl.ds(j * 128, 128)], sem)
            for j in range(_IC)
        ]
        for c in copies:
            c.wait()
        pltpu.sync_copy(rows_v, out_hbm.at[pl.ds(wid * _BPW, _BPW)])

    return _sc_gather


def _dense_body(frames_ref, g_ref, sel_ref, wf_ref, bf_ref, out_ref):
    vis = jnp.dot(frames_ref[...], wf_ref[...],
                  preferred_element_type=jnp.float32)
    g = g_ref[...]
    j = sel_ref[...] >> 17
    k = j >> 1
    half = jnp.where(k >= 2, g[:, 64:], g[:, :64])
    quarter = jnp.where((k & 1) == 1, half[:, 32:], half[:, :32])
    u = lax.bitcast_convert_type(quarter, jnp.uint32)
    val = jnp.where((j & 1) == 0, u & jnp.uint32(0xFFFF0000), u << 16)
    emb = lax.bitcast_convert_type(val, jnp.float32)
    out_ref[...] = vis + emb[:, :N_ACTIONS] + bf_ref[...]


_BLK = 2048


def _dense(frames, g, sel, w_f, b_f):
    return pl.pallas_call(
        _dense_body,
        grid=(B // _BLK,),
        in_specs=[
            pl.BlockSpec((_BLK, D_FRAME), lambda i: (i, 0)),
            pl.BlockSpec((_BLK, 128), lambda i: (i, 0)),
            pl.BlockSpec((_BLK, 1), lambda i: (i, 0)),
            pl.BlockSpec((D_FRAME, N_ACTIONS), lambda i: (0, 0)),
            pl.BlockSpec((1, N_ACTIONS), lambda i: (0, 0)),
        ],
        out_specs=pl.BlockSpec((_BLK, N_ACTIONS), lambda i: (i, 0)),
        out_shape=jax.ShapeDtypeStruct((B, N_ACTIONS), jnp.float32),
    )(frames, g, sel, w_f, b_f)


def kernel(frames, object_index, W_vis, b_vis, emb_table, W_pol, b_pol):
    idx = object_index.astype(jnp.int32)
    table_t = emb_table.T
    wb32 = jnp.pad(W_pol[D_VIS:, :], ((0, 0), (0, 32 - N_ACTIONS)))
    wbd = jax.scipy.linalg.block_diag(wb32, wb32, wb32, wb32)
    p2 = _repack(table_t, wbd)
    g = _make_sc_gather()(idx.reshape(B // 128, 128), p2)
    w_f = W_vis @ W_pol[:D_VIS, :]
    b_f = (b_vis @ W_pol[:D_VIS, :] + b_pol).reshape(1, N_ACTIONS)
    return _dense(frames, g, idx.reshape(B, 1), w_f, b_f)


# speedup vs baseline: 4.2236x; 1.0026x over previous
import functools

import jax
import jax.numpy as jnp
from jax import lax
from jax.experimental import pallas as pl
from jax.experimental.pallas import tpu as pltpu
from jax.experimental.pallas import tpu_sc as plsc

B = 16384
D_FRAME = 128
D_VIS = 64
D_EMB = 64
N_ACTIONS = 18
VOCAB = 1000000
REGION = 131072
P_ROWS = REGION

_NC = 2
_NS = 16
_NW = _NC * _NS
_BPW = B // _NW
_IC = _BPW // 128

_QBLK = 8192
_LAST_LHS_BLK = (VOCAB - 1) // _QBLK


def _bf16_bits(x):
    b = lax.bitcast_convert_type(x.astype(jnp.bfloat16), jnp.uint16)
    return b.astype(jnp.uint32)


def _repack_body(t0_ref, t1_ref, t2_ref, t3_ref, t4_ref, t5_ref, t6_ref,
                 t7_ref, wbd_ref, out_ref):
    wbd = wbd_ref[...]
    t_hi = jnp.concatenate(
        [t0_ref[...], t2_ref[...], t4_ref[...], t6_ref[...]], axis=0)
    t_lo = jnp.concatenate(
        [t1_ref[...], t3_ref[...], t5_ref[...], t7_ref[...]], axis=0)
    res_hi = lax.dot_general(t_hi, wbd, (((0,), (0,)), ((), ())),
                             preferred_element_type=jnp.float32)
    res_lo = lax.dot_general(t_lo, wbd, (((0,), (0,)), ((), ())),
                             preferred_element_type=jnp.float32)
    packed = (_bf16_bits(res_hi) << 16) | _bf16_bits(res_lo)
    out_ref[...] = lax.bitcast_convert_type(packed, jnp.float32)


def _repack(table_t, wbd):
    def lhs_spec(j):
        return pl.BlockSpec(
            (D_EMB, _QBLK),
            lambda i, j=j: (0, jnp.minimum(i + (REGION // _QBLK) * j,
                                           _LAST_LHS_BLK)))

    return pl.pallas_call(
        _repack_body,
        grid=(REGION // _QBLK,),
        in_specs=[
            lhs_spec(0), lhs_spec(1), lhs_spec(2), lhs_spec(3),
            lhs_spec(4), lhs_spec(5), lhs_spec(6), lhs_spec(7),
            pl.BlockSpec((4 * D_EMB, 128), lambda i: (0, 0)),
        ],
        out_specs=pl.BlockSpec((_QBLK, 128), lambda i: (i, 0)),
        out_shape=jax.ShapeDtypeStruct((P_ROWS, 128), jnp.float32),
    )(table_t, table_t, table_t, table_t, table_t, table_t, table_t, table_t,
      wbd)


@functools.cache
def _make_sc_gather():
    @functools.partial(
        pl.kernel,
        mesh=plsc.VectorSubcoreMesh(core_axis_name="c", subcore_axis_name="s"),
        out_type=jax.ShapeDtypeStruct((B, 128), jnp.float32),
        scratch_types=[
            pltpu.VMEM((_IC, 128), jnp.int32),
            pltpu.VMEM((_IC, 128), jnp.int32),
            pltpu.VMEM((_BPW, 128), jnp.float32),
            pltpu.SemaphoreType.DMA,
        ],
    )
    def _sc_gather(idx_hbm, p_hbm, out_hbm, idx_v, idx2_v, rows_v, sem):
        wid = lax.axis_index("s") * _NC + lax.axis_index("c")
        base = wid * _IC
        pltpu.sync_copy(idx_hbm.at[pl.ds(base, _IC)], idx_v)
        for j in range(_IC):
            for k in range(8):
                sl = pl.ds(k * 16, 16)
                idx2_v[j, sl] = idx_v[j, sl] & (REGION - 1)
        copies = [
            pltpu.async_copy(p_hbm.at[idx2_v.at[j]],
                             rows_v.at[pl.ds(j * 128, 128)], sem)
            for j in range(_IC)
        ]
        for c in copies:
            c.wait()
        pltpu.sync_copy(rows_v, out_hbm.at[pl.ds(wid * _BPW, _BPW)])

    return _sc_gather


def _dense_body(frames_ref, g_ref, sel_ref, wf_ref, bf_ref, out_ref):
    vis = jnp.dot(frames_ref[...], wf_ref[...],
                  preferred_element_type=jnp.float32)
    g = g_ref[...]
    j = sel_ref[...] >> 17
    k = j >> 1
    half = jnp.where(k >= 2, g[:, 64:], g[:, :64])
    quarter = jnp.where((k & 1) == 1, half[:, 32:], half[:, :32])
    u = lax.bitcast_convert_type(quarter, jnp.uint32)
    val = jnp.where((j & 1) == 0, u & jnp.uint32(0xFFFF0000), u << 16)
    emb = lax.bitcast_convert_type(val, jnp.float32)
    out_ref[...] = vis + emb[:, :N_ACTIONS] + bf_ref[...]


_BLK = 2048


def _dense(frames, g, sel, w_f, b_f):
    return pl.pallas_call(
        _dense_body,
        grid=(B // _BLK,),
        in_specs=[
            pl.BlockSpec((_BLK, D_FRAME), lambda i: (i, 0)),
            pl.BlockSpec((_BLK, 128), lambda i: (i, 0)),
            pl.BlockSpec((_BLK, 1), lambda i: (i, 0)),
            pl.BlockSpec((D_FRAME, N_ACTIONS), lambda i: (0, 0)),
            pl.BlockSpec((1, N_ACTIONS), lambda i: (0, 0)),
        ],
        out_specs=pl.BlockSpec((_BLK, N_ACTIONS), lambda i: (i, 0)),
        out_shape=jax.ShapeDtypeStruct((B, N_ACTIONS), jnp.float32),
    )(frames, g, sel, w_f, b_f)


def kernel(frames, object_index, W_vis, b_vis, emb_table, W_pol, b_pol):
    idx = object_index.astype(jnp.int32)
    table_t = emb_table.T
    wb32 = jnp.pad(W_pol[D_VIS:, :], ((0, 0), (0, 32 - N_ACTIONS)))
    wbd = jax.scipy.linalg.block_diag(wb32, wb32, wb32, wb32)
    p2 = _repack(table_t, wbd)
    g = _make_sc_gather()(idx.reshape(B // 128, 128), p2)
    wt = W_pol[:D_VIS, :]
    w_f = jnp.dot(W_vis, wt, precision=lax.Precision.HIGHEST)
    b_f = (jnp.dot(b_vis, wt, precision=lax.Precision.HIGHEST)
           + b_pol).reshape(1, N_ACTIONS)
    return _dense(frames, g, idx.reshape(B, 1), w_f, b_f)


# speedup vs baseline: 4.2239x; 1.0001x over previous
import functools

import jax
import jax.numpy as jnp
from jax import lax
from jax.experimental import pallas as pl
from jax.experimental.pallas import tpu as pltpu
from jax.experimental.pallas import tpu_sc as plsc

B = 16384
D_FRAME = 128
D_VIS = 64
D_EMB = 64
N_ACTIONS = 18
VOCAB = 1000000
REGION = 131072
P_ROWS = REGION

_NC = 2
_NS = 16
_NW = _NC * _NS
_BPW = B // _NW
_IC = _BPW // 128

_QBLK = 8192
_LAST_LHS_BLK = (VOCAB - 1) // _QBLK


def _bf16_bits(x):
    b = lax.bitcast_convert_type(x.astype(jnp.bfloat16), jnp.uint16)
    return b.astype(jnp.uint32)


def _repack_body(t0_ref, t1_ref, t2_ref, t3_ref, t4_ref, t5_ref, t6_ref,
                 t7_ref, wbd_ref, out_ref):
    wbd = wbd_ref[...]
    t_hi = jnp.concatenate(
        [t0_ref[...], t2_ref[...], t4_ref[...], t6_ref[...]], axis=0)
    t_lo = jnp.concatenate(
        [t1_ref[...], t3_ref[...], t5_ref[...], t7_ref[...]], axis=0)
    res_hi = lax.dot_general(t_hi, wbd, (((0,), (0,)), ((), ())),
                             preferred_element_type=jnp.float32)
    res_lo = lax.dot_general(t_lo, wbd, (((0,), (0,)), ((), ())),
                             preferred_element_type=jnp.float32)
    packed = (_bf16_bits(res_hi) << 16) | _bf16_bits(res_lo)
    out_ref[...] = lax.bitcast_convert_type(packed, jnp.float32)


def _repack(table_t, wbd):
    def lhs_spec(j):
        return pl.BlockSpec(
            (D_EMB, _QBLK),
            lambda i, j=j: (0, jnp.minimum(i + (REGION // _QBLK) * j,
                                           _LAST_LHS_BLK)))

    return pl.pallas_call(
        _repack_body,
        grid=(REGION // _QBLK,),
        in_specs=[
            lhs_spec(0), lhs_spec(1), lhs_spec(2), lhs_spec(3),
            lhs_spec(4), lhs_spec(5), lhs_spec(6), lhs_spec(7),
            pl.BlockSpec((4 * D_EMB, 128), lambda i: (0, 0)),
        ],
        out_specs=pl.BlockSpec((_QBLK, 128), lambda i: (i, 0)),
        out_shape=jax.ShapeDtypeStruct((P_ROWS, 128), jnp.float32),
    )(table_t, table_t, table_t, table_t, table_t, table_t, table_t, table_t,
      wbd)


@functools.cache
def _make_sc_gather():
    @functools.partial(
        pl.kernel,
        mesh=plsc.VectorSubcoreMesh(core_axis_name="c", subcore_axis_name="s"),
        out_type=jax.ShapeDtypeStruct((B, 128), jnp.float32),
        scratch_types=[
            pltpu.VMEM((_IC, 128), jnp.int32),
            pltpu.VMEM((_IC, 128), jnp.int32),
            pltpu.VMEM((_BPW, 128), jnp.float32),
            pltpu.SemaphoreType.DMA,
        ],
    )
    def _sc_gather(idx_hbm, p_hbm, out_hbm, idx_v, idx2_v, rows_v, sem):
        wid = lax.axis_index("s") * _NC + lax.axis_index("c")
        base = wid * _IC
        pltpu.sync_copy(idx_hbm.at[pl.ds(base, _IC)], idx_v)
        for j in range(_IC):
            for k in range(8):
                sl = pl.ds(k * 16, 16)
                idx2_v[j, sl] = idx_v[j, sl] & (REGION - 1)
        copies = [
            pltpu.async_copy(p_hbm.at[idx2_v.at[j]],
                             rows_v.at[pl.ds(j * 128, 128)], sem)
            for j in range(_IC)
        ]
        for c in copies:
            c.wait()
        pltpu.sync_copy(rows_v, out_hbm.at[pl.ds(wid * _BPW, _BPW)])

    return _sc_gather


def _dense_body(frames_ref, g_ref, sel_ref, wvis_ref, bvis_ref, wpt_ref,
                bpol_ref, out_ref):
    vis = jnp.dot(frames_ref[...], wvis_ref[...],
                  preferred_element_type=jnp.float32) + bvis_ref[...]
    g = g_ref[...]
    j = sel_ref[...] >> 17
    k = j >> 1
    half = jnp.where(k >= 2, g[:, 64:], g[:, :64])
    quarter = jnp.where((k & 1) == 1, half[:, 32:], half[:, :32])
    u = lax.bitcast_convert_type(quarter, jnp.uint32)
    val = jnp.where((j & 1) == 0, u & jnp.uint32(0xFFFF0000), u << 16)
    emb = lax.bitcast_convert_type(val, jnp.float32)
    out_ref[...] = (
        jnp.dot(vis, wpt_ref[...], preferred_element_type=jnp.float32)
        + emb[:, :N_ACTIONS]
        + bpol_ref[...]
    )


_BLK = 2048


def _dense(frames, g, sel, W_vis, b_vis2, wpt, b_pol2):
    return pl.pallas_call(
        _dense_body,
        grid=(B // _BLK,),
        in_specs=[
            pl.BlockSpec((_BLK, D_FRAME), lambda i: (i, 0)),
            pl.BlockSpec((_BLK, 128), lambda i: (i, 0)),
            pl.BlockSpec((_BLK, 1), lambda i: (i, 0)),
            pl.BlockSpec((D_FRAME, D_VIS), lambda i: (0, 0)),
            pl.BlockSpec((1, D_VIS), lambda i: (0, 0)),
            pl.BlockSpec((D_VIS, N_ACTIONS), lambda i: (0, 0)),
            pl.BlockSpec((1, N_ACTIONS), lambda i: (0, 0)),
        ],
        out_specs=pl.BlockSpec((_BLK, N_ACTIONS), lambda i: (i, 0)),
        out_shape=jax.ShapeDtypeStruct((B, N_ACTIONS), jnp.float32),
    )(frames, g, sel, W_vis, b_vis2, wpt, b_pol2)


def kernel(frames, object_index, W_vis, b_vis, emb_table, W_pol, b_pol):
    idx = object_index.astype(jnp.int32)
    table_t = emb_table.T
    wb32 = jnp.pad(W_pol[D_VIS:, :], ((0, 0), (0, 32 - N_ACTIONS)))
    wbd = jax.scipy.linalg.block_diag(wb32, wb32, wb32, wb32)
    p2 = _repack(table_t, wbd)
    g = _make_sc_gather()(idx.reshape(B // 128, 128), p2)
    return _dense(frames, g, idx.reshape(B, 1), W_vis,
                  b_vis.reshape(1, D_VIS), W_pol[:D_VIS, :],
                  b_pol.reshape(1, N_ACTIONS))
